# baseline (speedup 1.0000x reference)
import functools

import jax
import jax.numpy as jnp
from jax import lax
from jax.experimental import pallas as pl
from jax.experimental.pallas import tpu as pltpu
from jax.experimental.pallas import tpu_sc as plsc

N_TILE = 4096


def _tc_body(x_ref, w_ref, b_ref, vals_ref, pmax_ref, pidx_ref, *, n):
    t = pl.program_id(0)
    vals = jnp.dot(x_ref[...], w_ref[...],
                   preferred_element_type=jnp.float32) + b_ref[...]
    vals_ref[...] = vals
    col = t * N_TILE + lax.broadcasted_iota(jnp.int32, vals.shape, 1)
    masked = jnp.where(col < n, vals, -jnp.inf)
    row_max = jnp.max(masked, axis=1)
    row_arg = jnp.min(jnp.where(masked == row_max[:, None], col, n), axis=1)
    pmax_ref[0, 0, :] = row_max
    pidx_ref[0, 0, :] = row_arg


def _sc_merge_body(pmax_hbm, pidx_hbm, out_hbm, vmax, vidx, vout, *, t_tiles):
    wid = lax.axis_index("s") * 2 + lax.axis_index("c")

    @pl.when(wid < 8)
    def _():
        base = wid * 16
        pltpu.sync_copy(pmax_hbm, vmax)
        pltpu.sync_copy(pidx_hbm, vidx)
        m = vmax[0, pl.ds(base, 16)]
        a = vidx[0, pl.ds(base, 16)]
        for t in range(1, t_tiles):
            v = vmax[t, pl.ds(base, 16)]
            i = vidx[t, pl.ds(base, 16)]
            upd = v > m
            m = jnp.where(upd, v, m)
            a = jnp.where(upd, i, a)
        vout[...] = a
        pltpu.sync_copy(vout, out_hbm.at[wid])


def kernel(x, W, b):
    bsz, d = x.shape
    n = W.shape[1]
    t_tiles = pl.cdiv(n, N_TILE)

    vals, pmax, pidx = pl.pallas_call(
        functools.partial(_tc_body, n=n),
        grid=(t_tiles,),
        in_specs=[
            pl.BlockSpec((bsz, d), lambda t: (0, 0)),
            pl.BlockSpec((d, N_TILE), lambda t: (0, t)),
            pl.BlockSpec((1, N_TILE), lambda t: (0, t)),
        ],
        out_specs=[
            pl.BlockSpec((bsz, N_TILE), lambda t: (0, t)),
            pl.BlockSpec((1, 1, bsz), lambda t: (t, 0, 0)),
            pl.BlockSpec((1, 1, bsz), lambda t: (t, 0, 0)),
        ],
        out_shape=[
            jax.ShapeDtypeStruct((bsz, n), jnp.float32),
            jax.ShapeDtypeStruct((t_tiles, 1, bsz), jnp.float32),
            jax.ShapeDtypeStruct((t_tiles, 1, bsz), jnp.int32),
        ],
        compiler_params=pltpu.CompilerParams(
            dimension_semantics=("arbitrary",)),
    )(x, W, b.reshape(1, n))

    mesh = plsc.VectorSubcoreMesh(core_axis_name="c", subcore_axis_name="s")
    sc_merge = functools.partial(
        pl.kernel,
        mesh=mesh,
        out_type=jax.ShapeDtypeStruct((8, 16), jnp.int32),
        scratch_types=[
            pltpu.VMEM((t_tiles, bsz), jnp.float32),
            pltpu.VMEM((t_tiles, bsz), jnp.int32),
            pltpu.VMEM((16,), jnp.int32),
        ],
    )(functools.partial(_sc_merge_body, t_tiles=t_tiles))

    actions = sc_merge(pmax.reshape(t_tiles, bsz),
                       pidx.reshape(t_tiles, bsz)).reshape(bsz)
    return (actions.astype(jnp.int64), vals)
